# baseline (speedup 1.0000x reference)
import functools

import jax
import jax.numpy as jnp
from jax import lax
from jax.experimental import pallas as pl
from jax.experimental.pallas import tpu as pltpu
from jax.experimental.pallas import tpu_sc as plsc

_BLANK = 0
_SEQ = 2048
_BATCH = 16
_ALPHA = 1024
_LANES = 16

_SBLK = 128
_S_SC = 512
_S_TC = _SEQ - _S_SC
_HROWS = _S_SC // 2
_GROW = 16


def _argmax_block(x_ref, o_ref):
    xb = x_ref[...]
    m = jnp.max(xb, axis=2, keepdims=True)
    idx = lax.broadcasted_iota(jnp.int32, xb.shape, 2).astype(jnp.float32)
    ml = jnp.min(jnp.where(xb == m, idx, float(_ALPHA)), axis=2)
    o_ref[...] = ml.astype(jnp.int32).T


def _argmax_tc(x):
    seq, batch, alpha = x.shape
    return pl.pallas_call(
        _argmax_block,
        grid=(_S_TC // _SBLK,),
        in_specs=[pl.BlockSpec((_SBLK, batch, alpha), lambda i: (i, 0, 0))],
        out_specs=pl.BlockSpec((batch, _SBLK), lambda i: (0, i)),
        out_shape=jax.ShapeDtypeStruct((batch, _S_TC), jnp.int32),
    )(x)


def _argmax_sc_body(x_hbm, o_hbm, buf0, buf1, out_v, sem0, sem1):
    wid = lax.axis_index("s") * 2 + lax.axis_index("c")
    b = wid % _BATCH
    h = wid // _BATCH
    s0 = _S_TC + h * _HROWS
    bufs = (buf0, buf1)
    sems = (sem0, sem1)
    lanes = lax.iota(jnp.int32, _LANES)
    lanesf = lanes.astype(jnp.float32)
    ngroups = _HROWS // _GROW

    def start(g, par):
        pltpu.make_async_copy(
            x_hbm.at[pl.ds(s0 + g * _GROW, _GROW), b], bufs[par], sems[par]
        ).start()

    def wait(par):
        pltpu.make_async_copy(
            x_hbm.at[pl.ds(s0, _GROW), b], bufs[par], sems[par]
        ).wait()

    def process_group(g, par):
        buf = bufs[par]

        def row(r, _):
            def chunk(c, carry):
                maxv, idxf, afv = carry
                v = buf[r, pl.ds(c * _LANES, _LANES)]
                gt = v > maxv
                return (
                    jnp.maximum(maxv, v),
                    jnp.where(gt, afv, idxf),
                    afv + float(_LANES),
                )

            init = (
                jnp.full((_LANES,), -jnp.inf, jnp.float32),
                jnp.zeros((_LANES,), jnp.float32),
                lanesf,
            )
            maxv, idxf, _ = lax.fori_loop(0, _ALPHA // _LANES, chunk, init)
            m = jnp.max(maxv)
            mi = jnp.min(jnp.where(maxv == m, idxf, float(_ALPHA)))
            mivec = jnp.full((_LANES,), mi).astype(jnp.int32)
            rg = jnp.full((_LANES,), g * _GROW + r, jnp.int32)
            plsc.store_scatter(out_v, [rg], mivec, mask=lanes == 0)
            return 0

        lax.fori_loop(0, _GROW, row, 0)

    start(0, 0)
    start(1, 1)

    def pair(p, _):
        for par in (0, 1):
            g = p * 2 + par
            wait(par)
            process_group(g, par)

            @pl.when(g + 2 < ngroups)
            def _():
                start(g + 2, par)
        return 0

    lax.fori_loop(0, ngroups // 2, pair, 0)
    pltpu.sync_copy(out_v, o_hbm.at[b, pl.ds(h * _HROWS, _HROWS)])


@functools.cache
def _argmax_sc():
    return pl.kernel(
        _argmax_sc_body,
        out_type=jax.ShapeDtypeStruct((_BATCH, _S_SC), jnp.int32),
        mesh=plsc.VectorSubcoreMesh(core_axis_name="c", subcore_axis_name="s"),
        compiler_params=pltpu.CompilerParams(needs_layout_passes=False),
        scratch_types=[
            pltpu.VMEM((_GROW, _ALPHA), jnp.float32),
            pltpu.VMEM((_GROW, _ALPHA), jnp.float32),
            pltpu.VMEM((_HROWS,), jnp.int32),
            pltpu.SemaphoreType.DMA,
            pltpu.SemaphoreType.DMA,
        ],
    )


def _collapse_body(mltc_hbm, mlsc_hbm, len_hbm, tok_hbm, lenout_hbm,
                   row_v, out_v, len_v, tmp_v):
    wid = lax.axis_index("s") * 2 + lax.axis_index("c")

    @pl.when(wid < _BATCH)
    def _():
        b = wid
        pltpu.sync_copy(mltc_hbm.at[b], row_v.at[pl.ds(0, _S_TC)])
        pltpu.sync_copy(mlsc_hbm.at[b], row_v.at[pl.ds(_S_TC, _S_SC)])
        pltpu.sync_copy(len_hbm, len_v)
        lanes = lax.iota(jnp.int32, _LANES)
        lenb = plsc.load_gather(len_v, [jnp.full((_LANES,), b, jnp.int32)])
        last = jnp.full((_LANES,), _LANES - 1, jnp.int32)

        def step(c, rt):
            base = c * _LANES
            out_v[pl.ds(base, _LANES)] = jnp.full((_LANES,), -1, jnp.int32)
            v = row_v[pl.ds(base, _LANES)]
            gpos = base + lanes
            prevv = plsc.load_gather(row_v, [jnp.maximum(gpos - 1, 0)])
            prevv = jnp.where(gpos == 0, _BLANK, prevv)
            keep = (v != _BLANK) & ((prevv == _BLANK) | (v != prevv)) & (gpos < lenb)
            cs = plsc.cumsum(keep.astype(jnp.int32))
            pos = rt + cs - 1
            dest = jnp.where(keep, pos, 0)
            plsc.store_scatter(out_v, [dest], v, mask=keep)
            tmp_v[...] = cs
            return rt + plsc.load_gather(tmp_v, [last])

        rt = lax.fori_loop(
            0, _SEQ // _LANES, step, jnp.zeros((_LANES,), jnp.int32)
        )
        pltpu.sync_copy(out_v, tok_hbm.at[b])
        tmp_v[...] = rt
        pltpu.sync_copy(tmp_v, lenout_hbm.at[b])


@functools.cache
def _collapse_sc():
    return pl.kernel(
        _collapse_body,
        out_type=[
            jax.ShapeDtypeStruct((_BATCH, _SEQ), jnp.int32),
            jax.ShapeDtypeStruct((_BATCH, _LANES), jnp.int32),
        ],
        mesh=plsc.VectorSubcoreMesh(core_axis_name="c", subcore_axis_name="s"),
        compiler_params=pltpu.CompilerParams(needs_layout_passes=False),
        scratch_types=[
            pltpu.VMEM((_SEQ,), jnp.int32),
            pltpu.VMEM((_SEQ,), jnp.int32),
            pltpu.VMEM((_LANES,), jnp.int32),
            pltpu.VMEM((_LANES,), jnp.int32),
        ],
    )


@jax.jit
def kernel(x, lengths):
    ml_tc = _argmax_tc(x)
    ml_sc = _argmax_sc()(x)
    tok, lenm = _collapse_sc()(ml_tc, ml_sc, lengths)
    return tok, lenm[:, 0]


# speedup vs baseline: 1.5980x; 1.5980x over previous
import functools

import jax
import jax.numpy as jnp
from jax import lax
from jax.experimental import pallas as pl
from jax.experimental.pallas import tpu as pltpu
from jax.experimental.pallas import tpu_sc as plsc

_BLANK = 0
_SEQ = 2048
_BATCH = 16
_ALPHA = 1024
_LANES = 16

_SBLK = 128
_S_SC = 512
_S_TC = _SEQ - _S_SC
_HROWS = _S_SC // 2
_GROW = 16


def _argmax_block(x_ref, o_ref):
    xb = x_ref[...]
    m = jnp.max(xb, axis=2, keepdims=True)
    idx = lax.broadcasted_iota(jnp.int32, xb.shape, 2).astype(jnp.float32)
    ml = jnp.min(jnp.where(xb == m, idx, float(_ALPHA)), axis=2)
    o_ref[...] = ml.astype(jnp.int32).T


def _argmax_tc(x):
    seq, batch, alpha = x.shape
    return pl.pallas_call(
        _argmax_block,
        grid=(_S_TC // _SBLK,),
        in_specs=[pl.BlockSpec((_SBLK, batch, alpha), lambda i: (i, 0, 0))],
        out_specs=pl.BlockSpec((batch, _SBLK), lambda i: (0, i)),
        out_shape=jax.ShapeDtypeStruct((batch, _S_TC), jnp.int32),
    )(x)


def _argmax_sc_body(x_hbm, o_hbm, buf0, buf1, out_v, sem0, sem1):
    wid = lax.axis_index("s") * 2 + lax.axis_index("c")
    b = wid % _BATCH
    h = wid // _BATCH
    s0 = _S_TC + h * _HROWS
    bufs = (buf0, buf1)
    sems = (sem0, sem1)
    lanes = lax.iota(jnp.int32, _LANES)
    lanesf = lanes.astype(jnp.float32)
    ngroups = _HROWS // _GROW

    def start(g, par):
        pltpu.make_async_copy(
            x_hbm.at[pl.ds(s0 + g * _GROW, _GROW), b], bufs[par], sems[par]
        ).start()

    def wait(par):
        pltpu.make_async_copy(
            x_hbm.at[pl.ds(s0, _GROW), b], bufs[par], sems[par]
        ).wait()

    def process_group(g, par):
        buf = bufs[par]

        @plsc.parallel_loop(0, _GROW, unroll=2)
        def row(r):
            maxv = jnp.full((_LANES,), -jnp.inf, jnp.float32)
            idxf = jnp.zeros((_LANES,), jnp.float32)
            for c in range(_ALPHA // _LANES):
                v = buf[r, pl.ds(c * _LANES, _LANES)]
                gt = v > maxv
                maxv = jnp.maximum(maxv, v)
                idxf = jnp.where(gt, lanesf + float(c * _LANES), idxf)
            m = jnp.max(maxv)
            mi = jnp.min(jnp.where(maxv == m, idxf, float(_ALPHA)))
            mivec = jnp.full((_LANES,), mi).astype(jnp.int32)
            rg = jnp.full((_LANES,), g * _GROW + r, jnp.int32)
            plsc.store_scatter(out_v, [rg], mivec, mask=lanes == 0)

    start(0, 0)
    start(1, 1)

    def pair(p, _):
        for par in (0, 1):
            g = p * 2 + par
            wait(par)
            process_group(g, par)

            @pl.when(g + 2 < ngroups)
            def _():
                start(g + 2, par)
        return 0

    lax.fori_loop(0, ngroups // 2, pair, 0)
    pltpu.sync_copy(out_v, o_hbm.at[b, pl.ds(h * _HROWS, _HROWS)])


@functools.cache
def _argmax_sc():
    return pl.kernel(
        _argmax_sc_body,
        out_type=jax.ShapeDtypeStruct((_BATCH, _S_SC), jnp.int32),
        mesh=plsc.VectorSubcoreMesh(core_axis_name="c", subcore_axis_name="s"),
        compiler_params=pltpu.CompilerParams(needs_layout_passes=False),
        scratch_types=[
            pltpu.VMEM((_GROW, _ALPHA), jnp.float32),
            pltpu.VMEM((_GROW, _ALPHA), jnp.float32),
            pltpu.VMEM((_HROWS,), jnp.int32),
            pltpu.SemaphoreType.DMA,
            pltpu.SemaphoreType.DMA,
        ],
    )


def _collapse_body(mltc_hbm, mlsc_hbm, len_hbm, tok_hbm, lenout_hbm,
                   row_v, out_v, len_v, tmp_v):
    wid = lax.axis_index("s") * 2 + lax.axis_index("c")

    @pl.when(wid < _BATCH)
    def _():
        b = wid
        pltpu.sync_copy(mltc_hbm.at[b], row_v.at[pl.ds(0, _S_TC)])
        pltpu.sync_copy(mlsc_hbm.at[b], row_v.at[pl.ds(_S_TC, _S_SC)])
        pltpu.sync_copy(len_hbm, len_v)
        lanes = lax.iota(jnp.int32, _LANES)
        lenb = plsc.load_gather(len_v, [jnp.full((_LANES,), b, jnp.int32)])
        last = jnp.full((_LANES,), _LANES - 1, jnp.int32)

        def step(c, rt):
            base = c * _LANES
            out_v[pl.ds(base, _LANES)] = jnp.full((_LANES,), -1, jnp.int32)
            v = row_v[pl.ds(base, _LANES)]
            gpos = base + lanes
            prevv = plsc.load_gather(row_v, [jnp.maximum(gpos - 1, 0)])
            prevv = jnp.where(gpos == 0, _BLANK, prevv)
            keep = (v != _BLANK) & ((prevv == _BLANK) | (v != prevv)) & (gpos < lenb)
            cs = plsc.cumsum(keep.astype(jnp.int32))
            pos = rt + cs - 1
            dest = jnp.where(keep, pos, 0)
            plsc.store_scatter(out_v, [dest], v, mask=keep)
            tmp_v[...] = cs
            return rt + plsc.load_gather(tmp_v, [last])

        rt = lax.fori_loop(
            0, _SEQ // _LANES, step, jnp.zeros((_LANES,), jnp.int32)
        )
        pltpu.sync_copy(out_v, tok_hbm.at[b])
        tmp_v[...] = rt
        pltpu.sync_copy(tmp_v, lenout_hbm.at[b])


@functools.cache
def _collapse_sc():
    return pl.kernel(
        _collapse_body,
        out_type=[
            jax.ShapeDtypeStruct((_BATCH, _SEQ), jnp.int32),
            jax.ShapeDtypeStruct((_BATCH, _LANES), jnp.int32),
        ],
        mesh=plsc.VectorSubcoreMesh(core_axis_name="c", subcore_axis_name="s"),
        compiler_params=pltpu.CompilerParams(needs_layout_passes=False),
        scratch_types=[
            pltpu.VMEM((_SEQ,), jnp.int32),
            pltpu.VMEM((_SEQ,), jnp.int32),
            pltpu.VMEM((_LANES,), jnp.int32),
            pltpu.VMEM((_LANES,), jnp.int32),
        ],
    )


@jax.jit
def kernel(x, lengths):
    ml_tc = _argmax_tc(x)
    ml_sc = _argmax_sc()(x)
    tok, lenm = _collapse_sc()(ml_tc, ml_sc, lengths)
    return tok, lenm[:, 0]


# speedup vs baseline: 1.7628x; 1.1031x over previous
import functools

import jax
import jax.numpy as jnp
from jax import lax
from jax.experimental import pallas as pl
from jax.experimental.pallas import tpu as pltpu
from jax.experimental.pallas import tpu_sc as plsc

_BLANK = 0
_SEQ = 2048
_BATCH = 16
_ALPHA = 1024
_LANES = 16
_SBLK = 128


def _argmax_block(x_ref, o_ref):
    xb = x_ref[...]
    m = jnp.max(xb, axis=2, keepdims=True)
    idx = lax.broadcasted_iota(jnp.int32, xb.shape, 2).astype(jnp.float32)
    ml = jnp.min(jnp.where(xb == m, idx, float(_ALPHA)), axis=2)
    o_ref[...] = ml.astype(jnp.int32).T


def _argmax_tc(x):
    seq, batch, alpha = x.shape
    return pl.pallas_call(
        _argmax_block,
        grid=(seq // _SBLK,),
        in_specs=[pl.BlockSpec((_SBLK, batch, alpha), lambda i: (i, 0, 0))],
        out_specs=pl.BlockSpec((batch, _SBLK), lambda i: (0, i)),
        out_shape=jax.ShapeDtypeStruct((batch, seq), jnp.int32),
    )(x)


def _splat(v, lane):
    return v.at[jnp.full((_LANES,), lane, jnp.int32)].get(
        mode="promise_in_bounds"
    )


def _collapse_body(ml_hbm, len_hbm, tok_hbm, lenout_hbm, row_v, out_v, len_v, tmp_v):
    wid = lax.axis_index("s") * 2 + lax.axis_index("c")

    @pl.when(wid < _BATCH)
    def _():
        b = wid
        pltpu.sync_copy(ml_hbm.at[b], row_v)
        pltpu.sync_copy(len_hbm, len_v)
        lanes = lax.iota(jnp.int32, _LANES)
        lane0 = lanes == 0
        prev_sel = jnp.maximum(lanes - 1, 0)
        lenb = plsc.load_gather(len_v, [jnp.full((_LANES,), b, jnp.int32)])

        def step(c, carry):
            rt, pv = carry
            base = c * _LANES
            out_v[pl.ds(base, _LANES)] = jnp.full((_LANES,), -1, jnp.int32)
            v = row_v[pl.ds(base, _LANES)]
            gpos = base + lanes
            shifted = v.at[prev_sel].get(mode="promise_in_bounds")
            prevv = jnp.where(lane0, pv, shifted)
            keep = (v != _BLANK) & ((prevv == _BLANK) | (v != prevv)) & (gpos < lenb)
            cs = plsc.cumsum(keep.astype(jnp.int32))
            pos = rt + cs - 1
            dest = jnp.where(keep, pos, 0)
            plsc.store_scatter(out_v, [dest], v, mask=keep)
            return rt + _splat(cs, _LANES - 1), _splat(v, _LANES - 1)

        rt, _ = lax.fori_loop(
            0,
            _SEQ // _LANES,
            step,
            (jnp.zeros((_LANES,), jnp.int32), jnp.full((_LANES,), _BLANK, jnp.int32)),
        )
        pltpu.sync_copy(out_v, tok_hbm.at[b])
        tmp_v[...] = rt
        pltpu.sync_copy(tmp_v, lenout_hbm.at[b])


@functools.cache
def _collapse_sc():
    return pl.kernel(
        _collapse_body,
        out_type=[
            jax.ShapeDtypeStruct((_BATCH, _SEQ), jnp.int32),
            jax.ShapeDtypeStruct((_BATCH, _LANES), jnp.int32),
        ],
        mesh=plsc.VectorSubcoreMesh(core_axis_name="c", subcore_axis_name="s"),
        compiler_params=pltpu.CompilerParams(needs_layout_passes=False),
        scratch_types=[
            pltpu.VMEM((_SEQ,), jnp.int32),
            pltpu.VMEM((_SEQ,), jnp.int32),
            pltpu.VMEM((_LANES,), jnp.int32),
            pltpu.VMEM((_LANES,), jnp.int32),
        ],
    )


@jax.jit
def kernel(x, lengths):
    ml = _argmax_tc(x)
    tok, lenm = _collapse_sc()(ml, lengths)
    return tok, lenm[:, 0]
